# Initial kernel scaffold; baseline (speedup 1.0000x reference)
import jax
import jax.numpy as jnp
from jax.experimental import pallas as pl


def kernel(x, edge_index, pos_link, neg_link, batch, W_pb, b_pb, W_nb, b_nb, W_pd, b_pd, W_nd, b_nd):
    raise NotImplementedError("write your pallas kernel here")


# speedup vs baseline: 4.5850x; 4.5850x over previous
import functools

import jax
import jax.numpy as jnp
from jax import lax
from jax.experimental import pallas as pl
from jax.experimental.pallas import tpu as pltpu
from jax.experimental.pallas import tpu_sc as plsc

_N = 10000
_EP = 160000
_E = 320000
_F = 128
_G = 64

_CHUNK = 128
_NTILES = 16
_NP = 10112
_NODE_CHUNKS = _NP // _CHUNK
_F32 = jnp.float32


def _sc_segment_sum(n_edges):
  n_chunks = n_edges // _CHUNK
  assert n_chunks * _CHUNK == n_edges

  mesh = plsc.VectorSubcoreMesh(
      core_axis_name="c", subcore_axis_name="s", num_cores=2,
      num_subcores=_NTILES)

  @functools.partial(
      pl.kernel,
      out_type=(
          jax.ShapeDtypeStruct((_NP, _F), _F32),
          jax.ShapeDtypeStruct((_NP, _F), _F32),
          jax.ShapeDtypeStruct((_NP,), _F32),
          jax.ShapeDtypeStruct((_NP,), _F32),
      ),
      mesh=mesh,
      scratch_types=[
          pltpu.VMEM((_CHUNK, _F), _F32),
          pltpu.VMEM((_CHUNK,), jnp.int32),
          pltpu.VMEM((_CHUNK,), jnp.int32),
          pltpu.VMEM((_CHUNK,), _F32),
          pltpu.VMEM((_CHUNK,), _F32),
          pltpu.VMEM_SHARED((_NP, _F), _F32),
          pltpu.VMEM_SHARED((_NP,), _F32),
          pltpu.SemaphoreType.DMA,
      ],
  )
  def k(t0, t1, s0, d0, s1, d1, o0, o1, c0, c1,
        rows, idx_s, idx_d, ones_v, zeros_v, accum, cacc, sem):
    c = lax.axis_index("c")
    s = lax.axis_index("s")

    def fill_row(r, carry):
      for j in range(_F // 16):
        rows[r, pl.ds(j * 16, 16)] = jnp.zeros((16,), _F32)
      return carry
    lax.fori_loop(0, _CHUNK, fill_row, 0)
    for j in range(_CHUNK // 16):
      ones_v[pl.ds(j * 16, 16)] = jnp.ones((16,), _F32)
      zeros_v[pl.ds(j * 16, 16)] = jnp.zeros((16,), _F32)

    def zero_chunk(u, carry):
      base = (s + _NTILES * u) * _CHUNK
      pltpu.sync_copy(rows, accum.at[pl.ds(base, _CHUNK)])
      pltpu.sync_copy(zeros_v, cacc.at[pl.ds(base, _CHUNK)])
      return carry

    n_zero = (_NODE_CHUNKS - s + _NTILES - 1) // _NTILES
    lax.fori_loop(0, n_zero, zero_chunk, 0)
    plsc.subcore_barrier()

    def scatter_pass(table, src, dst):
      def body(t, carry):
        base = (s + _NTILES * t) * _CHUNK
        pltpu.sync_copy(src.at[pl.ds(base, _CHUNK)], idx_s)
        pltpu.sync_copy(dst.at[pl.ds(base, _CHUNK)], idx_d)
        pltpu.async_copy(table.at[idx_s], rows, sem).wait()
        pltpu.sync_copy(rows, accum.at[idx_d], add=True)
        pltpu.sync_copy(ones_v, cacc.at[idx_d], add=True)
        return carry
      n_iter = (n_chunks - s + _NTILES - 1) // _NTILES
      lax.fori_loop(0, n_iter, body, 0)

    @pl.when(c == 0)
    def _():
      scatter_pass(t0, s0, d0)

    @pl.when(c == 1)
    def _():
      scatter_pass(t1, s1, d1)

    plsc.subcore_barrier()

    def writeback(out_ref, cnt_ref):
      def wchunk(u, carry):
        base = (s + _NTILES * u) * _CHUNK
        pltpu.sync_copy(accum.at[pl.ds(base, _CHUNK)],
                        out_ref.at[pl.ds(base, _CHUNK)])
        pltpu.sync_copy(cacc.at[pl.ds(base, _CHUNK)],
                        cnt_ref.at[pl.ds(base, _CHUNK)])
        return carry
      lax.fori_loop(0, n_zero, wchunk, 0)

    @pl.when(c == 0)
    def _():
      writeback(o0, c0)

    @pl.when(c == 1)
    def _():
      writeback(o1, c1)

  return k


_BLK = 1000


def _tc1_body(x, sp, sn, aux, wpb, wnb, wpd, wnd, bias,
              hp, hn, ap, an, ssp, ssn):
  cp = jnp.maximum(aux[:, 0:1], 1.0)
  cn = jnp.maximum(aux[:, 1:2], 1.0)
  xb = x[...]
  mp = sp[...] / cp
  mn = sn[...] / cn
  dot = functools.partial(jnp.dot, preferred_element_type=_F32)
  h_p = jnp.tanh(dot(mp, wpb[0:_F, :]) + dot(xb, wpb[_F:, :]) + bias[0:1, :])
  h_n = jnp.tanh(dot(mn, wnb[0:_F, :]) + dot(xb, wnb[_F:, :]) + bias[1:2, :])
  hp[...] = h_p
  hn[...] = h_n
  ap[...] = dot(h_p, wpd[0:_F, :]) + dot(h_n, wpd[_F:2 * _F, :])
  an[...] = dot(h_p, wnd[0:_F, :]) + dot(h_n, wnd[_F:2 * _F, :])
  ssp[...] = (dot(h_p, wpd[4 * _F:5 * _F, :]) + dot(h_n, wpd[5 * _F:, :])
              + bias[2:3, :])
  ssn[...] = (dot(h_p, wnd[4 * _F:5 * _F, :]) + dot(h_n, wnd[5 * _F:, :])
              + bias[3:4, :])


def _tc1_call(x, sum_p, sum_n, aux, wpb, wnb, wpd_mid, wnd_mid, bias):
  row = lambda i: (i, 0)
  whole = lambda i: (0, 0)
  blk = pl.BlockSpec((_BLK, _F), row)
  out = jax.ShapeDtypeStruct((_N, _F), _F32)
  return pl.pallas_call(
      _tc1_body,
      grid=(_N // _BLK,),
      in_specs=[
          blk, blk, blk,
          pl.BlockSpec((_BLK, 8), row),
          pl.BlockSpec((2 * _F, _F), whole),
          pl.BlockSpec((2 * _F, _F), whole),
          pl.BlockSpec((6 * _F, _F), whole),
          pl.BlockSpec((6 * _F, _F), whole),
          pl.BlockSpec((8, _F), whole),
      ],
      out_specs=[blk] * 6,
      out_shape=[out] * 6,
  )(x, sum_p, sum_n, aux, wpb, wnb, wpd_mid, wnd_mid, bias)


def _tc2_body(s1p, s1n, s2p, s2n, ssp, ssn, aux, wpd2, wnd2, out,
              acc_p, acc_n, acc_c):
  i = pl.program_id(0)

  @pl.when(i == 0)
  def _():
    acc_p[...] = jnp.zeros_like(acc_p)
    acc_n[...] = jnp.zeros_like(acc_n)
    acc_c[...] = jnp.zeros_like(acc_c)

  cp = jnp.maximum(aux[:, 0:1], 1.0)
  cn = jnp.maximum(aux[:, 1:2], 1.0)
  ce = jnp.maximum(aux[:, 2:3], 1.0)
  bat = aux[:, 3:4]
  dot = functools.partial(jnp.dot, preferred_element_type=_F32)
  m2p = s2p[...] / ce
  m2n = s2n[...] / ce
  hp1 = jnp.tanh(s1p[...] / cp + dot(m2p, wpd2[0:_F, :])
                 + dot(m2n, wpd2[_F:, :]) + ssp[...])
  hn1 = jnp.tanh(s1n[...] / cn + dot(m2p, wnd2[0:_F, :])
                 + dot(m2n, wnd2[_F:, :]) + ssn[...])

  seg = lax.broadcasted_iota(jnp.int32, (1, _G), 1).astype(_F32)
  p_onehot = (bat == seg).astype(_F32)
  tdot = lambda a, b: lax.dot_general(
      a, b, dimension_numbers=(((0,), (0,)), ((), ())),
      preferred_element_type=_F32)
  acc_p[...] += tdot(p_onehot, hp1)
  acc_n[...] += tdot(p_onehot, hn1)
  acc_c[...] += tdot(p_onehot, jnp.ones((_BLK, _F), _F32))

  @pl.when(i == _N // _BLK - 1)
  def _():
    denom = jnp.maximum(acc_c[...], 1.0)
    out[:, 0:_F] = acc_p[...] / denom
    out[:, _F:] = acc_n[...] / denom


def _tc2_call(s1p, s1n, s2p, s2n, ssp, ssn, aux, wpd2, wnd2):
  row = lambda i: (i, 0)
  whole = lambda i: (0, 0)
  blk = pl.BlockSpec((_BLK, _F), row)
  return pl.pallas_call(
      _tc2_body,
      grid=(_N // _BLK,),
      in_specs=[
          blk, blk, blk, blk, blk, blk,
          pl.BlockSpec((_BLK, 8), row),
          pl.BlockSpec((2 * _F, _F), whole),
          pl.BlockSpec((2 * _F, _F), whole),
      ],
      out_specs=pl.BlockSpec((_G, 2 * _F), whole),
      out_shape=jax.ShapeDtypeStruct((_G, 2 * _F), _F32),
      scratch_shapes=[
          pltpu.VMEM((_G, _F), _F32),
          pltpu.VMEM((_G, _F), _F32),
          pltpu.VMEM((_G, _F), _F32),
      ],
  )(s1p, s1n, s2p, s2n, ssp, ssn, aux, wpd2, wnd2)


@jax.jit
def kernel(x, edge_index, pos_link, neg_link, batch,
           W_pb, b_pb, W_nb, b_nb, W_pd, b_pd, W_nd, b_nd):
  ps = pos_link[:, 0]
  pd = pos_link[:, 1]
  ns = neg_link[:, 0]
  nd = neg_link[:, 1]
  es = edge_index[0]
  ed = edge_index[1]

  sum_p, sum_n, cnt_p, cnt_n = _sc_segment_sum(_EP)(x, x, ps, pd, ns, nd)

  zeros_col = jnp.zeros((_NP, 1), _F32)
  aux1 = jnp.concatenate(
      [cnt_p[:, None], cnt_n[:, None]] + [zeros_col] * 6, axis=1)
  bias1 = jnp.concatenate(
      [b_pb[None, :], b_nb[None, :], b_pd[None, :], b_nd[None, :],
       jnp.zeros((4, _F), _F32)], axis=0)

  h_p, h_n, a_p, a_n, s_p, s_n = _tc1_call(
      x, sum_p, sum_n, aux1, W_pb, W_nb, W_pd, W_nd, bias1)

  sum1_p, sum1_n, _, _ = _sc_segment_sum(_EP)(a_p, a_n, ps, pd, ns, nd)
  s2p, s2n, cnt_e, _ = _sc_segment_sum(_E)(h_p, h_n, es, ed, es, ed)

  batch_f = jnp.concatenate(
      [batch.astype(_F32), jnp.zeros((_NP - _N,), _F32)])
  aux2 = jnp.concatenate(
      [cnt_p[:, None], cnt_n[:, None], cnt_e[:, None],
       batch_f[:, None]] + [zeros_col] * 4, axis=1)

  return _tc2_call(sum1_p, sum1_n, s2p, s2n, s_p, s_n, aux2,
                   W_pd[2 * _F:4 * _F, :], W_nd[2 * _F:4 * _F, :])


# speedup vs baseline: 5.3621x; 1.1695x over previous
import functools

import jax
import jax.numpy as jnp
from jax import lax
from jax.experimental import pallas as pl
from jax.experimental.pallas import tpu as pltpu
from jax.experimental.pallas import tpu_sc as plsc

_N = 10000
_EP = 160000
_E = 320000
_F = 128
_G = 64

_CHUNK = 128
_NTILES = 16
_NP = 10112
_NODE_CHUNKS = _NP // _CHUNK
_F32 = jnp.float32


_NBUF = 2
_IBUF = 2 * _NBUF


def _sc_segment_sum(n_edges, want_cnt0, want_cnt1):
  n_chunks = n_edges // _CHUNK
  assert n_chunks * _CHUNK == n_edges

  mesh = plsc.VectorSubcoreMesh(
      core_axis_name="c", subcore_axis_name="s", num_cores=2,
      num_subcores=_NTILES)

  @functools.partial(
      pl.kernel,
      out_type=(
          jax.ShapeDtypeStruct((_NP, _F), _F32),
          jax.ShapeDtypeStruct((_NP, _F), _F32),
          jax.ShapeDtypeStruct((_NP,), _F32),
          jax.ShapeDtypeStruct((_NP,), _F32),
      ),
      mesh=mesh,
      scratch_types=[
          pltpu.VMEM((_CHUNK, _F), _F32),
          pltpu.VMEM((2, _CHUNK), jnp.int32),
          pltpu.VMEM((_CHUNK,), _F32),
          pltpu.VMEM((_CHUNK,), _F32),
          pltpu.VMEM_SHARED((_NP, _F), _F32),
          pltpu.VMEM_SHARED((_NP,), _F32),
          pltpu.SemaphoreType.DMA,
      ],
  )
  def k(t0, t1, sd0, sd1, o0, o1, c0, c1,
        rows, idx2, ones_v, zeros_v, accum, cacc, sem):
    c = lax.axis_index("c")
    s = lax.axis_index("s")

    def fill_row(r, carry):
      for j in range(_F // 16):
        rows[r, pl.ds(j * 16, 16)] = jnp.zeros((16,), _F32)
      return carry
    lax.fori_loop(0, _CHUNK, fill_row, 0)
    for j in range(_CHUNK // 16):
      ones_v[pl.ds(j * 16, 16)] = jnp.ones((16,), _F32)
      zeros_v[pl.ds(j * 16, 16)] = jnp.zeros((16,), _F32)

    n_zero = (_NODE_CHUNKS - s + _NTILES - 1) // _NTILES

    def full_pass(table, sd, out_ref, cnt_ref, want_cnt):
      def zero_chunk(u, carry):
        base = (s + _NTILES * u) * _CHUNK
        pltpu.sync_copy(rows, accum.at[pl.ds(base, _CHUNK)])
        if want_cnt:
          pltpu.sync_copy(zeros_v, cacc.at[pl.ds(base, _CHUNK)])
        return carry
      lax.fori_loop(0, n_zero, zero_chunk, 0)
      plsc.subcore_barrier()

      def body(t, carry):
        base = (s + _NTILES * t) * _CHUNK
        pltpu.sync_copy(sd.at[:, pl.ds(base, _CHUNK)], idx2)
        pltpu.async_copy(table.at[idx2.at[0]], rows, sem).wait()
        pltpu.sync_copy(rows, accum.at[idx2.at[1]], add=True)
        if want_cnt:
          pltpu.sync_copy(ones_v, cacc.at[idx2.at[1]], add=True)
        return carry
      n_iter = (n_chunks - s + _NTILES - 1) // _NTILES
      lax.fori_loop(0, n_iter, body, 0)
      plsc.subcore_barrier()

      def wchunk(u, carry):
        base = (s + _NTILES * u) * _CHUNK
        pltpu.sync_copy(accum.at[pl.ds(base, _CHUNK)],
                        out_ref.at[pl.ds(base, _CHUNK)])
        if want_cnt:
          pltpu.sync_copy(cacc.at[pl.ds(base, _CHUNK)],
                          cnt_ref.at[pl.ds(base, _CHUNK)])
        return carry
      lax.fori_loop(0, n_zero, wchunk, 0)

    @pl.when(c == 0)
    def _():
      full_pass(t0, sd0, o0, c0, want_cnt0)

    @pl.when(c == 1)
    def _():
      full_pass(t1, sd1, o1, c1, want_cnt1)

  return k


_BLK = 1000


def _tc1_body(x, sp, sn, aux, wpb, wnb, wpd, wnd, bias,
              hp, hn, ap, an, ssp, ssn):
  cp = jnp.maximum(aux[:, 0:1], 1.0)
  cn = jnp.maximum(aux[:, 1:2], 1.0)
  xb = x[...]
  mp = sp[...] / cp
  mn = sn[...] / cn
  dot = functools.partial(jnp.dot, preferred_element_type=_F32)
  h_p = jnp.tanh(dot(mp, wpb[0:_F, :]) + dot(xb, wpb[_F:, :]) + bias[0:1, :])
  h_n = jnp.tanh(dot(mn, wnb[0:_F, :]) + dot(xb, wnb[_F:, :]) + bias[1:2, :])
  hp[...] = h_p
  hn[...] = h_n
  ap[...] = dot(h_p, wpd[0:_F, :]) + dot(h_n, wpd[_F:2 * _F, :])
  an[...] = dot(h_p, wnd[0:_F, :]) + dot(h_n, wnd[_F:2 * _F, :])
  ssp[...] = (dot(h_p, wpd[4 * _F:5 * _F, :]) + dot(h_n, wpd[5 * _F:, :])
              + bias[2:3, :])
  ssn[...] = (dot(h_p, wnd[4 * _F:5 * _F, :]) + dot(h_n, wnd[5 * _F:, :])
              + bias[3:4, :])


def _tc1_call(x, sum_p, sum_n, aux, wpb, wnb, wpd_mid, wnd_mid, bias):
  row = lambda i: (i, 0)
  whole = lambda i: (0, 0)
  blk = pl.BlockSpec((_BLK, _F), row)
  out = jax.ShapeDtypeStruct((_N, _F), _F32)
  return pl.pallas_call(
      _tc1_body,
      grid=(_N // _BLK,),
      in_specs=[
          blk, blk, blk,
          pl.BlockSpec((_BLK, 8), row),
          pl.BlockSpec((2 * _F, _F), whole),
          pl.BlockSpec((2 * _F, _F), whole),
          pl.BlockSpec((6 * _F, _F), whole),
          pl.BlockSpec((6 * _F, _F), whole),
          pl.BlockSpec((8, _F), whole),
      ],
      out_specs=[blk] * 6,
      out_shape=[out] * 6,
  )(x, sum_p, sum_n, aux, wpb, wnb, wpd_mid, wnd_mid, bias)


def _tc2_body(s1p, s1n, s2p, s2n, ssp, ssn, aux, wpd2, wnd2, out,
              acc_p, acc_n, acc_c):
  i = pl.program_id(0)

  @pl.when(i == 0)
  def _():
    acc_p[...] = jnp.zeros_like(acc_p)
    acc_n[...] = jnp.zeros_like(acc_n)
    acc_c[...] = jnp.zeros_like(acc_c)

  cp = jnp.maximum(aux[:, 0:1], 1.0)
  cn = jnp.maximum(aux[:, 1:2], 1.0)
  ce = jnp.maximum(aux[:, 2:3], 1.0)
  bat = aux[:, 3:4]
  dot = functools.partial(jnp.dot, preferred_element_type=_F32)
  m2p = s2p[...] / ce
  m2n = s2n[...] / ce
  hp1 = jnp.tanh(s1p[...] / cp + dot(m2p, wpd2[0:_F, :])
                 + dot(m2n, wpd2[_F:, :]) + ssp[...])
  hn1 = jnp.tanh(s1n[...] / cn + dot(m2p, wnd2[0:_F, :])
                 + dot(m2n, wnd2[_F:, :]) + ssn[...])

  seg = lax.broadcasted_iota(jnp.int32, (1, _G), 1).astype(_F32)
  p_onehot = (bat == seg).astype(_F32)
  tdot = lambda a, b: lax.dot_general(
      a, b, dimension_numbers=(((0,), (0,)), ((), ())),
      preferred_element_type=_F32)
  acc_p[...] += tdot(p_onehot, hp1)
  acc_n[...] += tdot(p_onehot, hn1)
  acc_c[...] += tdot(p_onehot, jnp.ones((_BLK, _F), _F32))

  @pl.when(i == _N // _BLK - 1)
  def _():
    denom = jnp.maximum(acc_c[...], 1.0)
    out[:, 0:_F] = acc_p[...] / denom
    out[:, _F:] = acc_n[...] / denom


def _tc2_call(s1p, s1n, s2p, s2n, ssp, ssn, aux, wpd2, wnd2):
  row = lambda i: (i, 0)
  whole = lambda i: (0, 0)
  blk = pl.BlockSpec((_BLK, _F), row)
  return pl.pallas_call(
      _tc2_body,
      grid=(_N // _BLK,),
      in_specs=[
          blk, blk, blk, blk, blk, blk,
          pl.BlockSpec((_BLK, 8), row),
          pl.BlockSpec((2 * _F, _F), whole),
          pl.BlockSpec((2 * _F, _F), whole),
      ],
      out_specs=pl.BlockSpec((_G, 2 * _F), whole),
      out_shape=jax.ShapeDtypeStruct((_G, 2 * _F), _F32),
      scratch_shapes=[
          pltpu.VMEM((_G, _F), _F32),
          pltpu.VMEM((_G, _F), _F32),
          pltpu.VMEM((_G, _F), _F32),
      ],
  )(s1p, s1n, s2p, s2n, ssp, ssn, aux, wpd2, wnd2)


@jax.jit
def kernel(x, edge_index, pos_link, neg_link, batch,
           W_pb, b_pb, W_nb, b_nb, W_pd, b_pd, W_nd, b_nd):
  pos_t = pos_link.T
  neg_t = neg_link.T

  sum_p, sum_n, cnt_p, cnt_n = _sc_segment_sum(_EP, True, True)(
      x, x, pos_t, neg_t)

  zeros_col = jnp.zeros((_NP, 1), _F32)
  aux1 = jnp.concatenate(
      [cnt_p[:, None], cnt_n[:, None]] + [zeros_col] * 6, axis=1)
  bias1 = jnp.concatenate(
      [b_pb[None, :], b_nb[None, :], b_pd[None, :], b_nd[None, :],
       jnp.zeros((4, _F), _F32)], axis=0)

  h_p, h_n, a_p, a_n, s_p, s_n = _tc1_call(
      x, sum_p, sum_n, aux1, W_pb, W_nb, W_pd, W_nd, bias1)

  sum1_p, sum1_n, _, _ = _sc_segment_sum(_EP, False, False)(
      a_p, a_n, pos_t, neg_t)
  s2p, s2n, cnt_e, _ = _sc_segment_sum(_E, True, False)(
      h_p, h_n, edge_index, edge_index)

  batch_f = jnp.concatenate(
      [batch.astype(_F32), jnp.zeros((_NP - _N,), _F32)])
  aux2 = jnp.concatenate(
      [cnt_p[:, None], cnt_n[:, None], cnt_e[:, None],
       batch_f[:, None]] + [zeros_col] * 4, axis=1)

  return _tc2_call(sum1_p, sum1_n, s2p, s2n, s_p, s_n, aux2,
                   W_pd[2 * _F:4 * _F, :], W_nd[2 * _F:4 * _F, :])
